# baseline (speedup 1.0000x reference)
import jax
import jax.numpy as jnp
from jax.experimental import pallas as pl
from jax.experimental.pallas import tpu as pltpu

EMBED_DIM = 32
BLOCK_COLS = 128


def _mf_body(user_hbm, item_hbm, users_hbm, items_hbm, out_ref,
             uidx, iidx, ublock, iblock, sem_ui, sem_u, sem_i):
    cu = pltpu.make_async_copy(user_hbm, uidx, sem_ui)
    ci = pltpu.make_async_copy(item_hbm, iidx, sem_ui)
    cu.start()
    ci.start()
    cu.wait()
    ci.wait()
    u = uidx[0]
    i = iidx[0]
    ub = pl.multiple_of((u // BLOCK_COLS) * BLOCK_COLS, BLOCK_COLS)
    ib = pl.multiple_of((i // BLOCK_COLS) * BLOCK_COLS, BLOCK_COLS)
    du = pltpu.make_async_copy(
        users_hbm.at[:, pl.ds(ub, BLOCK_COLS)], ublock, sem_u)
    di = pltpu.make_async_copy(
        items_hbm.at[:, pl.ds(ib, BLOCK_COLS)], iblock, sem_i)
    du.start()
    di.start()
    du.wait()
    di.wait()
    lanes = jax.lax.broadcasted_iota(jnp.int32, (EMBED_DIM, BLOCK_COLS), 1)
    ucol = jnp.sum(jnp.where(lanes == u % BLOCK_COLS, ublock[...], 0.0),
                   axis=1, keepdims=True)
    icol = jnp.sum(jnp.where(lanes == i % BLOCK_COLS, iblock[...], 0.0),
                   axis=1, keepdims=True)
    out_ref[...] = jnp.sum(ucol * icol, axis=0, keepdims=True)


def kernel(user, item, users_emb, items_emb):
    out = pl.pallas_call(
        _mf_body,
        in_specs=[
            pl.BlockSpec(memory_space=pl.ANY),
            pl.BlockSpec(memory_space=pl.ANY),
            pl.BlockSpec(memory_space=pl.ANY),
            pl.BlockSpec(memory_space=pl.ANY),
        ],
        out_specs=pl.BlockSpec((1, 1), lambda: (0, 0)),
        out_shape=jax.ShapeDtypeStruct((1, 1), jnp.float32),
        scratch_shapes=[
            pltpu.SMEM((1,), jnp.int32),
            pltpu.SMEM((1,), jnp.int32),
            pltpu.VMEM((EMBED_DIM, BLOCK_COLS), jnp.float32),
            pltpu.VMEM((EMBED_DIM, BLOCK_COLS), jnp.float32),
            pltpu.SemaphoreType.DMA,
            pltpu.SemaphoreType.DMA,
            pltpu.SemaphoreType.DMA,
        ],
    )(user.reshape(1), item.reshape(1), users_emb.T, items_emb.T)
    return out[0, 0]


# speedup vs baseline: 1.2025x; 1.2025x over previous
import jax
import jax.numpy as jnp
from jax.experimental import pallas as pl
from jax.experimental.pallas import tpu as pltpu

EMBED_DIM = 32
BLOCK_COLS = 128


def _mf_body(uidx_ref, iidx_ref, ublock_ref, iblock_ref, out_ref):
    u = uidx_ref[...] % BLOCK_COLS
    i = iidx_ref[...] % BLOCK_COLS
    lanes = jax.lax.broadcasted_iota(jnp.int32, (EMBED_DIM, BLOCK_COLS), 1)
    ucol = jnp.sum(jnp.where(lanes == u, ublock_ref[...], 0.0),
                   axis=1, keepdims=True)
    icol = jnp.sum(jnp.where(lanes == i, iblock_ref[...], 0.0),
                   axis=1, keepdims=True)
    out_ref[...] = jnp.sum(ucol * icol, axis=0, keepdims=True)


def kernel(user, item, users_emb, items_emb):
    out = pl.pallas_call(
        _mf_body,
        grid_spec=pltpu.PrefetchScalarGridSpec(
            num_scalar_prefetch=2,
            grid=(1,),
            in_specs=[
                pl.BlockSpec((EMBED_DIM, BLOCK_COLS),
                             lambda g, uref, iref: (0, uref[...] // BLOCK_COLS)),
                pl.BlockSpec((EMBED_DIM, BLOCK_COLS),
                             lambda g, uref, iref: (0, iref[...] // BLOCK_COLS)),
            ],
            out_specs=pl.BlockSpec((1, 1), lambda g, uref, iref: (0, 0)),
        ),
        out_shape=jax.ShapeDtypeStruct((1, 1), jnp.float32),
    )(user, item, users_emb.T, items_emb.T)
    return out[0, 0]
